# baseline (speedup 1.0000x reference)
import jax
import jax.numpy as jnp
from jax import lax
from jax.experimental import pallas as pl
from jax.experimental.pallas import tpu as pltpu
from jax.experimental.pallas import tpu_sc as plsc

B = 1024
N = 100000
D = 128

NUM_CORES = 2
NUM_SUBCORES = 16
L = 16
SUB = 128
K = 2
BIG = K * SUB
ROWS_MAIN = 6400
N_BIG_MAIN = ROWS_MAIN // BIG
LAST_BASE = 15 * ROWS_MAIN
N_BIG_LAST = (N - LAST_BASE) // BIG
EXTRA_BASE = LAST_BASE + N_BIG_LAST * BIG
TAIL = N - EXTRA_BASE - SUB
IDX_TROWS = 50
SEGS_PER_TILE = B // NUM_SUBCORES


def _sc_body(xu, bu, xn, bn, means_out,
             xbuf_a, xbuf_b, xbuf_c, idx_all, idxbuf_t, cnt2, lcnt, cbuf,
             csum, dbuf, sem_a, sem_b, sem_c, sem_sa, sem_sb, sem_sc,
             acc, cnt_stage):
    c = lax.axis_index("c")
    s = lax.axis_index("s")

    def zd(i, carry):
        for j in range(D // L):
            dbuf[i, pl.ds(L * j, L)] = jnp.zeros((L,), jnp.float32)
        return carry

    lax.fori_loop(0, SEGS_PER_TILE, zd, None)

    def zc(i, carry):
        cnt2[pl.ds(L * i, L)] = jnp.zeros((L,), jnp.float32)
        return carry

    lax.fori_loop(0, L * B // L, zc, None)

    seg0 = s * SEGS_PER_TILE
    pltpu.sync_copy(dbuf, acc.at[pl.ds(seg0, SEGS_PER_TILE)])
    plsc.subcore_barrier()

    lane_off = lax.iota(jnp.int32, L) * B
    ones = jnp.ones((L,), jnp.float32)

    def process(x_hbm, idx_hbm, n_sub, with_tail):
        base = s * ROWS_MAIN
        pltpu.sync_copy(idx_hbm.at[s], idx_all)

        xbs = (xbuf_a, xbuf_b, xbuf_c)
        sems = (sem_a, sem_b, sem_c)

        def start_load(t, j):
            pltpu.async_copy(x_hbm.at[pl.ds(base + j * SUB, SUB), :],
                             xbs[t], sems[t])

        def wait_load(t):
            pltpu.make_async_copy(x_hbm.at[pl.ds(0, SUB), :],
                                  xbs[t], sems[t]).wait()

        def counts(row, groups):
            for g in range(groups):
                idx16 = idx_all[row, pl.ds(L * g, L)]
                plsc.addupdate_scatter(cnt2, [lane_off + idx16], ones)

        for t in range(3):
            start_load(t, t)

        n_full_rows = (31 if with_tail else IDX_TROWS)

        def count_row(row, carry):
            counts(row, SUB // L)
            return carry

        lax.fori_loop(0, n_full_rows, count_row, None)
        if with_tail:
            counts(31, TAIL // L)

        ssems = (sem_sa, sem_sb, sem_sc)

        def wait_scat(t):
            pltpu.make_async_copy(xbs[t], acc.at[idx_all.at[0]],
                                  ssems[t]).wait()

        def ring(g, carry):
            for t in range(3):
                j = 3 * g + t
                wait_load(t)
                pltpu.async_copy(xbs[t], acc.at[idx_all.at[j]],
                                 ssems[t], add=True)
                tp = (t + 2) % 3
                jp = j - 1

                @pl.when(jp >= 0)
                def _():
                    wait_scat(tp)

                    @pl.when(jp + 3 < n_sub)
                    def _():
                        start_load(tp, jp + 3)

            return carry

        lax.fori_loop(0, n_sub // 3, ring, None)
        wait_scat(2)
        for t in range(n_sub % 3):
            j = (n_sub // 3) * 3 + t
            wait_load(t)
            pltpu.sync_copy(xbs[t], acc.at[idx_all.at[j]], add=True)

        if with_tail:
            t0 = LAST_BASE + 31 * SUB
            for g in range(TAIL // L):
                idxbuf_t[pl.ds(L * g, L)] = idx_all[31, pl.ds(L * g, L)]
            pltpu.sync_copy(x_hbm.at[pl.ds(t0, TAIL), :],
                            xbuf_a.at[pl.ds(0, TAIL), :])
            pltpu.sync_copy(xbuf_a.at[pl.ds(0, TAIL), :],
                            acc.at[idxbuf_t], add=True)

    @pl.when(jnp.logical_and(c == 0, s < NUM_SUBCORES - 1))
    def _():
        process(xu, bu, 50, False)

    @pl.when(jnp.logical_and(c == 0, s == NUM_SUBCORES - 1))
    def _():
        process(xu, bu, 31, True)

    @pl.when(jnp.logical_and(c == 1, s < NUM_SUBCORES - 1))
    def _():
        process(xn, bn, 50, False)

    @pl.when(jnp.logical_and(c == 1, s == NUM_SUBCORES - 1))
    def _():
        process(xn, bn, 31, True)

    def lane_reduce(k, carry):
        tot = jnp.zeros((L,), jnp.float32)
        for r in range(L):
            tot = tot + cnt2[pl.ds(r * B + L * k, L)]
        lcnt[pl.ds(L * k, L)] = tot
        return carry

    lax.fori_loop(0, B // L, lane_reduce, None)
    pltpu.sync_copy(lcnt, cnt_stage.at[s])
    plsc.subcore_barrier()

    pltpu.sync_copy(cnt_stage, cbuf)
    for k in range(SEGS_PER_TILE // L):
        tot = jnp.zeros((L,), jnp.float32)
        for r in range(NUM_SUBCORES):
            tot = tot + cbuf[r, pl.ds(seg0 + L * k, L)]
        csum[pl.ds(L * k, L)] = tot

    pltpu.sync_copy(acc.at[pl.ds(seg0, SEGS_PER_TILE)], dbuf)

    def norm_group(m, carry):
        inv = 1.0 / jnp.maximum(csum[pl.ds(L * m, L)], 1.0)
        for r in range(L):
            k = L * m + r
            scale = jnp.full((L,), inv[r])
            for j in range(D // L):
                dbuf[k, pl.ds(L * j, L)] = dbuf[k, pl.ds(L * j, L)] * scale
        return carry

    lax.fori_loop(0, SEGS_PER_TILE // L, norm_group, None)
    out0 = c * B + seg0
    pltpu.sync_copy(dbuf, means_out.at[pl.ds(out0, SEGS_PER_TILE)])


def _segment_means(x_user, batch_user2d, x_news, batch_news2d):
    mesh = plsc.VectorSubcoreMesh(core_axis_name="c", subcore_axis_name="s")
    fn = pl.kernel(
        _sc_body,
        mesh=mesh,
        compiler_params=pltpu.CompilerParams(needs_layout_passes=False),
        out_type=jax.ShapeDtypeStruct((NUM_CORES * B, D), jnp.float32),
        scratch_types=[
            pltpu.VMEM((SUB, D), jnp.float32),
            pltpu.VMEM((SUB, D), jnp.float32),
            pltpu.VMEM((SUB, D), jnp.float32),
            pltpu.VMEM((IDX_TROWS, SUB), jnp.int32),
            pltpu.VMEM((TAIL,), jnp.int32),
            pltpu.VMEM((L * B,), jnp.float32),
            pltpu.VMEM((B,), jnp.float32),
            pltpu.VMEM((NUM_SUBCORES, B), jnp.float32),
            pltpu.VMEM((SEGS_PER_TILE,), jnp.float32),
            pltpu.VMEM((SEGS_PER_TILE, D), jnp.float32),
            pltpu.SemaphoreType.DMA,
            pltpu.SemaphoreType.DMA,
            pltpu.SemaphoreType.DMA,
            pltpu.SemaphoreType.DMA,
            pltpu.SemaphoreType.DMA,
            pltpu.SemaphoreType.DMA,
            pltpu.VMEM_SHARED((B, D), jnp.float32),
            pltpu.VMEM_SHARED((NUM_SUBCORES, B), jnp.float32),
        ],
    )
    return fn(x_user, batch_user2d, x_news, batch_news2d)


def _mlp_body(means_ref, ne_ref, w1_ref, b1_ref, w2_ref, b2_ref,
              w3_ref, b3_ref, out_ref):
    pu = means_ref[0:B, :]
    pn = means_ref[B:2 * B, :]
    hp = jax.lax.Precision.HIGHEST
    h = jnp.dot(pu, w1_ref[0:D, :], precision=hp)
    h = h + jnp.dot(pn, w1_ref[D:2 * D, :], precision=hp)
    h = jnp.maximum(h + b1_ref[0:1, :], 0.0)
    h = jnp.maximum(jnp.dot(h, w2_ref[...], precision=hp) + b2_ref[0:1, :], 0.0)
    out_ref[...] = (jnp.dot(h, w3_ref[...], precision=hp) + b3_ref[0:1, :]
                    + ne_ref[...])


def _ids_3d(batch):
    ids = batch.astype(jnp.int32)
    pad = NUM_SUBCORES * IDX_TROWS * SUB - N
    return jnp.pad(ids, (0, pad)).reshape(NUM_SUBCORES, IDX_TROWS, SUB)


def kernel(x_user, batch_user, x_news, batch_news, news_embeddings,
           W1, b1, W2, b2, W3, b3):
    means = _segment_means(x_user, _ids_3d(batch_user),
                           x_news, _ids_3d(batch_news))
    return pl.pallas_call(
        _mlp_body,
        out_shape=jax.ShapeDtypeStruct((B, D), jnp.float32),
    )(means, news_embeddings,
      W1, b1.reshape(1, D), W2, b2.reshape(1, D), W3, b3.reshape(1, D))


# speedup vs baseline: 1.1249x; 1.1249x over previous
import jax
import jax.numpy as jnp
from jax import lax
from jax.experimental import pallas as pl
from jax.experimental.pallas import tpu as pltpu
from jax.experimental.pallas import tpu_sc as plsc

B = 1024
N = 100000
D = 128

NUM_CORES = 2
NUM_SUBCORES = 16
L = 16
SUB = 128
K = 2
BIG = K * SUB
ROWS_MAIN = 6400
N_BIG_MAIN = ROWS_MAIN // BIG
LAST_BASE = 15 * ROWS_MAIN
N_BIG_LAST = (N - LAST_BASE) // BIG
EXTRA_BASE = LAST_BASE + N_BIG_LAST * BIG
TAIL = N - EXTRA_BASE - SUB
IDX_TROWS = 50
SEGS_PER_TILE = B // NUM_SUBCORES


def _sc_body(xu, bu, xn, bn, means_out,
             xbuf_a, xbuf_b, xbuf_c, xbuf_d, idx_all, idxbuf_t, cnt2, lcnt,
             cbuf, csum, dbuf, sem_a, sem_b, sem_c, sem_d, acc, cnt_stage):
    c = lax.axis_index("c")
    s = lax.axis_index("s")

    def zd(i, carry):
        for j in range(D // L):
            dbuf[i, pl.ds(L * j, L)] = jnp.zeros((L,), jnp.float32)
        return carry

    lax.fori_loop(0, SEGS_PER_TILE, zd, None)

    def zc(i, carry):
        cnt2[pl.ds(L * i, L)] = jnp.zeros((L,), jnp.float32)
        return carry

    lax.fori_loop(0, L * B // L, zc, None)

    seg0 = s * SEGS_PER_TILE
    pltpu.sync_copy(dbuf, acc.at[pl.ds(seg0, SEGS_PER_TILE)])
    plsc.subcore_barrier()

    lane_off = lax.iota(jnp.int32, L) * B
    ones = jnp.ones((L,), jnp.float32)

    def process(x_hbm, idx_hbm, n_sub, with_tail):
        base = s * ROWS_MAIN
        pltpu.sync_copy(idx_hbm.at[s], idx_all)

        xbs = (xbuf_a, xbuf_b, xbuf_c, xbuf_d)
        sems = (sem_a, sem_b, sem_c, sem_d)

        def start_load(t, j):
            pltpu.async_copy(x_hbm.at[pl.ds(base + j * SUB, SUB), :],
                             xbs[t], sems[t])

        def wait_load(t):
            pltpu.make_async_copy(x_hbm.at[pl.ds(0, SUB), :],
                                  xbs[t], sems[t]).wait()

        def counts(row, groups):
            for g in range(groups):
                idx16 = idx_all[row, pl.ds(L * g, L)]
                plsc.addupdate_scatter(cnt2, [lane_off + idx16], ones)

        for t in range(4):
            start_load(t, t)

        n_full_rows = (31 if with_tail else IDX_TROWS)

        def count_row(row, carry):
            counts(row, SUB // L)
            return carry

        lax.fori_loop(0, n_full_rows, count_row, None)
        if with_tail:
            counts(31, TAIL // L)

        def ring(g, carry):
            for t in range(4):
                j = 4 * g + t
                wait_load(t)
                pltpu.sync_copy(xbs[t], acc.at[idx_all.at[j]], add=True)

                @pl.when(j + 4 < n_sub)
                def _():
                    start_load(t, j + 4)

            return carry

        lax.fori_loop(0, n_sub // 4, ring, None)
        for t in range(n_sub % 4):
            j = (n_sub // 4) * 4 + t
            wait_load(t)
            pltpu.sync_copy(xbs[t], acc.at[idx_all.at[j]], add=True)

        if with_tail:
            t0 = LAST_BASE + 31 * SUB
            for g in range(TAIL // L):
                idxbuf_t[pl.ds(L * g, L)] = idx_all[31, pl.ds(L * g, L)]
            pltpu.sync_copy(x_hbm.at[pl.ds(t0, TAIL), :],
                            xbuf_a.at[pl.ds(0, TAIL), :])
            pltpu.sync_copy(xbuf_a.at[pl.ds(0, TAIL), :],
                            acc.at[idxbuf_t], add=True)

    @pl.when(jnp.logical_and(c == 0, s < NUM_SUBCORES - 1))
    def _():
        process(xu, bu, 50, False)

    @pl.when(jnp.logical_and(c == 0, s == NUM_SUBCORES - 1))
    def _():
        process(xu, bu, 31, True)

    @pl.when(jnp.logical_and(c == 1, s < NUM_SUBCORES - 1))
    def _():
        process(xn, bn, 50, False)

    @pl.when(jnp.logical_and(c == 1, s == NUM_SUBCORES - 1))
    def _():
        process(xn, bn, 31, True)

    def lane_reduce(k, carry):
        tot = jnp.zeros((L,), jnp.float32)
        for r in range(L):
            tot = tot + cnt2[pl.ds(r * B + L * k, L)]
        lcnt[pl.ds(L * k, L)] = tot
        return carry

    lax.fori_loop(0, B // L, lane_reduce, None)
    pltpu.sync_copy(lcnt, cnt_stage.at[s])
    plsc.subcore_barrier()

    pltpu.sync_copy(cnt_stage, cbuf)
    for k in range(SEGS_PER_TILE // L):
        tot = jnp.zeros((L,), jnp.float32)
        for r in range(NUM_SUBCORES):
            tot = tot + cbuf[r, pl.ds(seg0 + L * k, L)]
        csum[pl.ds(L * k, L)] = tot

    pltpu.sync_copy(acc.at[pl.ds(seg0, SEGS_PER_TILE)], dbuf)

    def norm_group(m, carry):
        inv = 1.0 / jnp.maximum(csum[pl.ds(L * m, L)], 1.0)
        for r in range(L):
            k = L * m + r
            scale = jnp.full((L,), inv[r])
            for j in range(D // L):
                dbuf[k, pl.ds(L * j, L)] = dbuf[k, pl.ds(L * j, L)] * scale
        return carry

    lax.fori_loop(0, SEGS_PER_TILE // L, norm_group, None)
    out0 = c * B + seg0
    pltpu.sync_copy(dbuf, means_out.at[pl.ds(out0, SEGS_PER_TILE)])


def _segment_means(x_user, batch_user2d, x_news, batch_news2d):
    mesh = plsc.VectorSubcoreMesh(core_axis_name="c", subcore_axis_name="s")
    fn = pl.kernel(
        _sc_body,
        mesh=mesh,
        compiler_params=pltpu.CompilerParams(needs_layout_passes=False),
        out_type=jax.ShapeDtypeStruct((NUM_CORES * B, D), jnp.float32),
        scratch_types=[
            pltpu.VMEM((SUB, D), jnp.float32),
            pltpu.VMEM((SUB, D), jnp.float32),
            pltpu.VMEM((SUB, D), jnp.float32),
            pltpu.VMEM((SUB, D), jnp.float32),
            pltpu.VMEM((IDX_TROWS, SUB), jnp.int32),
            pltpu.VMEM((TAIL,), jnp.int32),
            pltpu.VMEM((L * B,), jnp.float32),
            pltpu.VMEM((B,), jnp.float32),
            pltpu.VMEM((NUM_SUBCORES, B), jnp.float32),
            pltpu.VMEM((SEGS_PER_TILE,), jnp.float32),
            pltpu.VMEM((SEGS_PER_TILE, D), jnp.float32),
            pltpu.SemaphoreType.DMA,
            pltpu.SemaphoreType.DMA,
            pltpu.SemaphoreType.DMA,
            pltpu.SemaphoreType.DMA,
            pltpu.VMEM_SHARED((B, D), jnp.float32),
            pltpu.VMEM_SHARED((NUM_SUBCORES, B), jnp.float32),
        ],
    )
    return fn(x_user, batch_user2d, x_news, batch_news2d)


def _mlp_body(means_ref, ne_ref, w1_ref, b1_ref, w2_ref, b2_ref,
              w3_ref, b3_ref, out_ref):
    pu = means_ref[0:B, :]
    pn = means_ref[B:2 * B, :]
    hp = jax.lax.Precision.HIGHEST
    h = jnp.dot(pu, w1_ref[0:D, :], precision=hp)
    h = h + jnp.dot(pn, w1_ref[D:2 * D, :], precision=hp)
    h = jnp.maximum(h + b1_ref[0:1, :], 0.0)
    h = jnp.maximum(jnp.dot(h, w2_ref[...], precision=hp) + b2_ref[0:1, :], 0.0)
    out_ref[...] = (jnp.dot(h, w3_ref[...], precision=hp) + b3_ref[0:1, :]
                    + ne_ref[...])


def _ids_3d(batch):
    ids = batch.astype(jnp.int32)
    pad = NUM_SUBCORES * IDX_TROWS * SUB - N
    return jnp.pad(ids, (0, pad)).reshape(NUM_SUBCORES, IDX_TROWS, SUB)


def kernel(x_user, batch_user, x_news, batch_news, news_embeddings,
           W1, b1, W2, b2, W3, b3):
    means = _segment_means(x_user, _ids_3d(batch_user),
                           x_news, _ids_3d(batch_news))
    return pl.pallas_call(
        _mlp_body,
        out_shape=jax.ShapeDtypeStruct((B, D), jnp.float32),
    )(means, news_embeddings,
      W1, b1.reshape(1, D), W2, b2.reshape(1, D), W3, b3.reshape(1, D))


# speedup vs baseline: 1.1262x; 1.0011x over previous
import jax
import jax.numpy as jnp
from jax import lax
from jax.experimental import pallas as pl
from jax.experimental.pallas import tpu as pltpu
from jax.experimental.pallas import tpu_sc as plsc

B = 1024
N = 100000
D = 128

NUM_CORES = 2
NUM_SUBCORES = 16
L = 16
SUB = 128
ROWS_MAIN = 6400
LAST_BASE = 15 * ROWS_MAIN
TAIL = N - LAST_BASE - 31 * SUB
IDX_TROWS = 50
SEGS_PER_TILE = B // NUM_SUBCORES


def _sc_body(xu, bu, xn, bn, means_out,
             xbuf_a, xbuf_b, xbuf_c, xbuf_d, idx_all, idxbuf_t, cnt2, lcnt,
             cbuf, csum, dbuf, sem_a, sem_b, sem_c, sem_d, acc, cnt_stage):
    c = lax.axis_index("c")
    s = lax.axis_index("s")

    def zd(i, carry):
        for j in range(D // L):
            dbuf[i, pl.ds(L * j, L)] = jnp.zeros((L,), jnp.float32)
        return carry

    lax.fori_loop(0, SEGS_PER_TILE, zd, None)

    def zc(i, carry):
        cnt2[pl.ds(L * i, L)] = jnp.zeros((L,), jnp.float32)
        return carry

    lax.fori_loop(0, L * B // L, zc, None)

    seg0 = s * SEGS_PER_TILE
    pltpu.sync_copy(dbuf, acc.at[pl.ds(seg0, SEGS_PER_TILE)])
    plsc.subcore_barrier()

    lane_off = lax.iota(jnp.int32, L) * B
    ones = jnp.ones((L,), jnp.float32)

    def process(x_hbm, idx_hbm, n_sub, with_tail):
        base = s * ROWS_MAIN
        pltpu.sync_copy(idx_hbm.at[s], idx_all)

        xbs = (xbuf_a, xbuf_b, xbuf_c, xbuf_d)
        sems = (sem_a, sem_b, sem_c, sem_d)

        def start_load(t, j):
            pltpu.async_copy(x_hbm.at[pl.ds(base + j * SUB, SUB), :],
                             xbs[t], sems[t])

        def wait_load(t):
            pltpu.make_async_copy(x_hbm.at[pl.ds(0, SUB), :],
                                  xbs[t], sems[t]).wait()

        def counts(row, groups):
            for g in range(groups):
                idx16 = idx_all[row, pl.ds(L * g, L)]
                plsc.addupdate_scatter(cnt2, [lane_off + idx16], ones)

        for t in range(4):
            start_load(t, t)

        n_full_rows = (31 if with_tail else IDX_TROWS)

        def count_row(row, carry):
            counts(row, SUB // L)
            return carry

        lax.fori_loop(0, n_full_rows, count_row, None)
        if with_tail:
            counts(31, TAIL // L)

        def ring(g, carry):
            for t in range(4):
                j = 4 * g + t
                wait_load(t)
                pltpu.sync_copy(xbs[t], acc.at[idx_all.at[j]], add=True)

                @pl.when(j + 4 < n_sub)
                def _():
                    start_load(t, j + 4)

            return carry

        lax.fori_loop(0, n_sub // 4, ring, None)
        for t in range(n_sub % 4):
            j = (n_sub // 4) * 4 + t
            wait_load(t)
            pltpu.sync_copy(xbs[t], acc.at[idx_all.at[j]], add=True)

        if with_tail:
            t0 = LAST_BASE + 31 * SUB
            for g in range(TAIL // L):
                idxbuf_t[pl.ds(L * g, L)] = idx_all[31, pl.ds(L * g, L)]
            pltpu.sync_copy(x_hbm.at[pl.ds(t0, TAIL), :],
                            xbuf_a.at[pl.ds(0, TAIL), :])
            pltpu.sync_copy(xbuf_a.at[pl.ds(0, TAIL), :],
                            acc.at[idxbuf_t], add=True)

    @pl.when(jnp.logical_and(c == 0, s < NUM_SUBCORES - 1))
    def _():
        process(xu, bu, 50, False)

    @pl.when(jnp.logical_and(c == 0, s == NUM_SUBCORES - 1))
    def _():
        process(xu, bu, 31, True)

    @pl.when(jnp.logical_and(c == 1, s < NUM_SUBCORES - 1))
    def _():
        process(xn, bn, 50, False)

    @pl.when(jnp.logical_and(c == 1, s == NUM_SUBCORES - 1))
    def _():
        process(xn, bn, 31, True)

    def lane_reduce(k, carry):
        tot = jnp.zeros((L,), jnp.float32)
        for r in range(L):
            tot = tot + cnt2[pl.ds(r * B + L * k, L)]
        lcnt[pl.ds(L * k, L)] = tot
        return carry

    lax.fori_loop(0, B // L, lane_reduce, None)
    pltpu.sync_copy(lcnt, cnt_stage.at[s])
    plsc.subcore_barrier()

    pltpu.sync_copy(cnt_stage, cbuf)
    for k in range(SEGS_PER_TILE // L):
        tot = jnp.zeros((L,), jnp.float32)
        for r in range(NUM_SUBCORES):
            tot = tot + cbuf[r, pl.ds(seg0 + L * k, L)]
        csum[pl.ds(L * k, L)] = tot

    pltpu.sync_copy(acc.at[pl.ds(seg0, SEGS_PER_TILE)], dbuf)

    def norm_group(m, carry):
        inv = 1.0 / jnp.maximum(csum[pl.ds(L * m, L)], 1.0)
        for r in range(L):
            k = L * m + r
            scale = jnp.full((L,), inv[r])
            for j in range(D // L):
                dbuf[k, pl.ds(L * j, L)] = dbuf[k, pl.ds(L * j, L)] * scale
        return carry

    lax.fori_loop(0, SEGS_PER_TILE // L, norm_group, None)
    out0 = c * B + seg0
    pltpu.sync_copy(dbuf, means_out.at[pl.ds(out0, SEGS_PER_TILE)])


def _segment_means(x_user, batch_user2d, x_news, batch_news2d):
    mesh = plsc.VectorSubcoreMesh(core_axis_name="c", subcore_axis_name="s")
    fn = pl.kernel(
        _sc_body,
        mesh=mesh,
        compiler_params=pltpu.CompilerParams(needs_layout_passes=False),
        out_type=jax.ShapeDtypeStruct((NUM_CORES * B, D), jnp.float32),
        scratch_types=[
            pltpu.VMEM((SUB, D), jnp.float32),
            pltpu.VMEM((SUB, D), jnp.float32),
            pltpu.VMEM((SUB, D), jnp.float32),
            pltpu.VMEM((SUB, D), jnp.float32),
            pltpu.VMEM((IDX_TROWS, SUB), jnp.int32),
            pltpu.VMEM((TAIL,), jnp.int32),
            pltpu.VMEM((L * B,), jnp.float32),
            pltpu.VMEM((B,), jnp.float32),
            pltpu.VMEM((NUM_SUBCORES, B), jnp.float32),
            pltpu.VMEM((SEGS_PER_TILE,), jnp.float32),
            pltpu.VMEM((SEGS_PER_TILE, D), jnp.float32),
            pltpu.SemaphoreType.DMA,
            pltpu.SemaphoreType.DMA,
            pltpu.SemaphoreType.DMA,
            pltpu.SemaphoreType.DMA,
            pltpu.VMEM_SHARED((B, D), jnp.float32),
            pltpu.VMEM_SHARED((NUM_SUBCORES, B), jnp.float32),
        ],
    )
    return fn(x_user, batch_user2d, x_news, batch_news2d)


def _mlp_body(means_ref, ne_ref, w1_ref, b1_ref, w2_ref, b2_ref,
              w3_ref, b3_ref, out_ref):
    pu = means_ref[0:B, :]
    pn = means_ref[B:2 * B, :]
    hp = jax.lax.Precision.HIGHEST
    h = jnp.dot(pu, w1_ref[0:D, :], precision=hp)
    h = h + jnp.dot(pn, w1_ref[D:2 * D, :], precision=hp)
    h = jnp.maximum(h + b1_ref[0:1, :], 0.0)
    h = jnp.maximum(jnp.dot(h, w2_ref[...], precision=hp) + b2_ref[0:1, :], 0.0)
    out_ref[...] = (jnp.dot(h, w3_ref[...], precision=hp) + b3_ref[0:1, :]
                    + ne_ref[...])


def _ids_3d(batch):
    ids = batch.astype(jnp.int32)
    pad = NUM_SUBCORES * IDX_TROWS * SUB - N
    return jnp.pad(ids, (0, pad)).reshape(NUM_SUBCORES, IDX_TROWS, SUB)


def kernel(x_user, batch_user, x_news, batch_news, news_embeddings,
           W1, b1, W2, b2, W3, b3):
    means = _segment_means(x_user, _ids_3d(batch_user),
                           x_news, _ids_3d(batch_news))
    return pl.pallas_call(
        _mlp_body,
        out_shape=jax.ShapeDtypeStruct((B, D), jnp.float32),
    )(means, news_embeddings,
      W1, b1.reshape(1, D), W2, b2.reshape(1, D), W3, b3.reshape(1, D))
